# baseline (speedup 1.0000x reference)
import functools

import jax
import jax.numpy as jnp
from jax import lax
from jax.experimental import pallas as pl
from jax.experimental.pallas import tpu as pltpu
from jax.experimental.pallas import tpu_sc as plsc

NUM_CORES = 2
NUM_SUBCORES = 16
NUM_WORKERS = NUM_CORES * NUM_SUBCORES
CHUNK = 128


@functools.partial(jax.jit, static_argnames=("chunks", "dim"))
def _sc_gather(idx3, weight, *, chunks, dim):
    n_rows = NUM_WORKERS * chunks * CHUNK
    mesh = plsc.VectorSubcoreMesh(
        core_axis_name="c", subcore_axis_name="s", num_cores=NUM_CORES
    )

    @functools.partial(
        pl.kernel,
        mesh=mesh,
        out_type=jax.ShapeDtypeStruct((n_rows, dim), jnp.float32),
        scratch_types=[
            pltpu.VMEM((chunks, CHUNK), jnp.int32),
            pltpu.VMEM((CHUNK, dim), jnp.float32),
            pltpu.SemaphoreType.DMA,
        ],
        compiler_params=pltpu.CompilerParams(use_tc_tiling_on_sc=False),
    )
    def k(idx_hbm, table_hbm, out_hbm, idx_v, rows_v, sem):
        wid = lax.axis_index("s") * NUM_CORES + lax.axis_index("c")
        pltpu.sync_copy(idx_hbm.at[wid], idx_v)
        base = wid * (chunks * CHUNK)

        def step(j, carry):
            pltpu.async_copy(table_hbm.at[idx_v.at[j]], rows_v, sem).wait()
            pltpu.sync_copy(rows_v, out_hbm.at[pl.ds(base + j * CHUNK, CHUNK)])
            return carry

        lax.fori_loop(0, chunks, step, 0)

    return k(idx3, weight)


def kernel(indices, weight):
    b0, b1 = indices.shape
    dim = weight.shape[1]
    n = b0 * b1
    per_w = n // NUM_WORKERS
    chunks = per_w // CHUNK
    idx3 = indices.astype(jnp.int32).reshape(NUM_WORKERS, chunks, CHUNK)
    out = _sc_gather(idx3, weight, chunks=chunks, dim=dim)
    return out.reshape(b0, b1, dim)


# speedup vs baseline: 1.0449x; 1.0449x over previous
import functools

import jax
import jax.numpy as jnp
from jax import lax
from jax.experimental import pallas as pl
from jax.experimental.pallas import tpu as pltpu
from jax.experimental.pallas import tpu_sc as plsc

NUM_CORES = 2
NUM_SUBCORES = 16
NUM_WORKERS = NUM_CORES * NUM_SUBCORES
CHUNK = 128
NBUF = 10


@functools.partial(jax.jit, static_argnames=("chunks", "dim"))
def _sc_gather(idx3, weight, *, chunks, dim):
    n_rows = NUM_WORKERS * chunks * CHUNK
    outer = chunks // NBUF
    mesh = plsc.VectorSubcoreMesh(
        core_axis_name="c", subcore_axis_name="s", num_cores=NUM_CORES
    )

    @functools.partial(
        pl.kernel,
        mesh=mesh,
        out_type=jax.ShapeDtypeStruct((n_rows, dim), jnp.float32),
        scratch_types=[
            pltpu.VMEM((chunks, CHUNK), jnp.int32),
            pltpu.VMEM((NBUF, CHUNK, dim), jnp.float32),
            [pltpu.SemaphoreType.DMA] * NBUF,
            [pltpu.SemaphoreType.DMA] * NBUF,
        ],
        compiler_params=pltpu.CompilerParams(use_tc_tiling_on_sc=False),
    )
    def k(idx_hbm, table_hbm, out_hbm, idx_v, rows_v, gsems, wsems):
        wid = lax.axis_index("s") * NUM_CORES + lax.axis_index("c")
        pltpu.sync_copy(idx_hbm.at[wid], idx_v)
        base = wid * (chunks * CHUNK)

        def start_gather(j, b):
            pltpu.async_copy(table_hbm.at[idx_v.at[j]], rows_v.at[b], gsems[b])

        def wait_gather(j, b):
            pltpu.make_async_copy(
                table_hbm.at[idx_v.at[j]], rows_v.at[b], gsems[b]
            ).wait()

        def start_write(j, b):
            pltpu.async_copy(
                rows_v.at[b], out_hbm.at[pl.ds(base + j * CHUNK, CHUNK)], wsems[b]
            )

        def wait_write(j, b):
            pltpu.make_async_copy(
                rows_v.at[b], out_hbm.at[pl.ds(base + j * CHUNK, CHUNK)], wsems[b]
            ).wait()

        for b in range(NBUF):
            start_gather(b, b)

        def body(i, carry):
            j0 = i * NBUF
            for b in range(NBUF):
                wait_gather(j0 + b, b)
                start_write(j0 + b, b)

            @pl.when(i < outer - 1)
            def _next():
                for b in range(NBUF):
                    wait_write(j0 + b, b)
                    start_gather(j0 + NBUF + b, b)

            return carry

        lax.fori_loop(0, outer, body, 0)

        last = (outer - 1) * NBUF
        for b in range(NBUF):
            wait_write(last + b, b)

    return k(idx3, weight)


def kernel(indices, weight):
    b0, b1 = indices.shape
    dim = weight.shape[1]
    n = b0 * b1
    per_w = n // NUM_WORKERS
    chunks = per_w // CHUNK
    idx3 = indices.astype(jnp.int32).reshape(NUM_WORKERS, chunks, CHUNK)
    out = _sc_gather(idx3, weight, chunks=chunks, dim=dim)
    return out.reshape(b0, b1, dim)
